# baseline (speedup 1.0000x reference)
import functools

import jax
import jax.numpy as jnp
from jax import lax
from jax.experimental import pallas as pl
from jax.experimental.pallas import tpu as pltpu
from jax.experimental.pallas import tpu_sc as plsc

D_MODEL = 1024
NCORES = 2
NSUB = 16
LANES = 16
NW = NCORES * NSUB
K = 16
NB = 3


def _encode_sc(x2d, tok, emb):
    n_tok = x2d.shape[0]
    tpw = n_tok // NW
    nch = tpw // K
    nsteps = (nch - 1) // NB
    mesh = plsc.VectorSubcoreMesh(core_axis_name="c", subcore_axis_name="s")

    @functools.partial(
        pl.kernel,
        out_type=jax.ShapeDtypeStruct((n_tok, D_MODEL), jnp.float32),
        mesh=mesh,
        scratch_types=[
            pltpu.VMEM((tpw,), jnp.int32)]
            + [pltpu.VMEM((K, D_MODEL), jnp.float32) for _ in range(2 * NB)]
            + [pltpu.SemaphoreType.DMA for _ in range(2 * NB)],
    )
    def k(x_hbm, tok_hbm, emb_hbm, out_hbm, idx_v, *bufs_and_sems):
        xbs = bufs_and_sems[:NB]
        rbs = bufs_and_sems[NB:2 * NB]
        sem_in = bufs_and_sems[2 * NB:3 * NB]
        sem_st = bufs_and_sems[3 * NB:4 * NB]

        wid = lax.axis_index("s") * NCORES + lax.axis_index("c")
        base = pl.multiple_of(wid * tpw, 8)
        pltpu.sync_copy(tok_hbm.at[pl.ds(base, tpw)], idx_v)

        def start_in(c, s):
            coff = pl.multiple_of(c * K, 8)
            row0 = pl.multiple_of(base + c * K, 8)
            pltpu.async_copy(emb_hbm.at[idx_v.at[pl.ds(coff, K)]],
                             rbs[s], sem_in[s])
            pltpu.async_copy(x_hbm.at[pl.ds(row0, K)], xbs[s], sem_in[s])

        def wait_in(s):
            pltpu.make_async_copy(emb_hbm.at[idx_v.at[pl.ds(0, K)]],
                                  rbs[s], sem_in[s]).wait()
            pltpu.make_async_copy(x_hbm.at[pl.ds(base, K)], xbs[s],
                                  sem_in[s]).wait()

        def start_st(c, s):
            row0 = pl.multiple_of(base + c * K, 8)
            pltpu.async_copy(rbs[s], out_hbm.at[pl.ds(row0, K)], sem_st[s])

        def wait_st(s):
            pltpu.make_async_copy(rbs[s], out_hbm.at[pl.ds(base, K)],
                                  sem_st[s]).wait()

        def add_chunk(s):
            @pl.loop(0, K)
            def _rows(t):
                @pl.loop(0, D_MODEL // LANES, unroll=8)
                def _add(j):
                    off = pl.multiple_of(j * LANES, LANES)
                    plsc.addupdate(rbs[s].at[t, pl.ds(off, LANES)],
                                   xbs[s][t, pl.ds(off, LANES)])

        start_in(0, 0)

        @pl.loop(0, nsteps)
        def _steps(i):
            for b in range(NB):
                c = i * NB + b
                s = b
                sn = (b + 1) % NB

                if b < 2:
                    @pl.when(i > 0)
                    def _():
                        wait_st(sn)
                else:
                    wait_st(sn)

                start_in(c + 1, sn)
                wait_in(s)
                add_chunk(s)
                start_st(c, s)

        last = nch - 1
        s_last = last % NB
        wait_in(s_last)
        add_chunk(s_last)
        start_st(last, s_last)
        for s in range(NB):
            wait_st(s)

    return k(x2d, tok, emb)


def kernel(x, tokens, emb_weight):
    b, l, d = x.shape
    x2d = x.reshape(b * l, d)
    tok = tokens.reshape(-1).astype(jnp.int32)
    out = _encode_sc(x2d, tok, emb_weight)
    return out.reshape(b, l, d)


# speedup vs baseline: 1.9096x; 1.9096x over previous
import functools

import jax
import jax.numpy as jnp
from jax import lax
from jax.experimental import pallas as pl
from jax.experimental.pallas import tpu as pltpu
from jax.experimental.pallas import tpu_sc as plsc

D_MODEL = 1024
NCORES = 2
NSUB = 16
LANES = 16
NW = NCORES * NSUB
K = 8
NB = 6
LA = 3


def _encode_sc(x2d, tok, emb):
    n_tok = x2d.shape[0]
    tpw = n_tok // NW
    nch = tpw // K
    nsteps = (nch - 2) // NB
    mesh = plsc.VectorSubcoreMesh(core_axis_name="c", subcore_axis_name="s")

    @functools.partial(
        pl.kernel,
        out_type=jax.ShapeDtypeStruct((n_tok, D_MODEL), jnp.float32),
        mesh=mesh,
        scratch_types=[
            pltpu.VMEM((tpw,), jnp.int32)]
            + [pltpu.VMEM((K, D_MODEL), jnp.float32) for _ in range(2 * NB)]
            + [pltpu.SemaphoreType.DMA for _ in range(2 * NB)],
    )
    def k(x_hbm, tok_hbm, emb_hbm, out_hbm, idx_v, *bufs_and_sems):
        xbs = bufs_and_sems[:NB]
        rbs = bufs_and_sems[NB:2 * NB]
        sem_in = bufs_and_sems[2 * NB:3 * NB]
        sem_st = bufs_and_sems[3 * NB:4 * NB]

        wid = lax.axis_index("s") * NCORES + lax.axis_index("c")
        base = pl.multiple_of(wid * tpw, 8)
        pltpu.sync_copy(tok_hbm.at[pl.ds(base, tpw)], idx_v)

        def start_in(c, s):
            coff = pl.multiple_of(c * K, 8)
            row0 = pl.multiple_of(base + c * K, 8)
            pltpu.async_copy(emb_hbm.at[idx_v.at[pl.ds(coff, K)]],
                             rbs[s], sem_in[s])
            pltpu.async_copy(x_hbm.at[pl.ds(row0, K)], xbs[s], sem_in[s])

        def wait_in(s):
            pltpu.make_async_copy(emb_hbm.at[idx_v.at[pl.ds(0, K)]],
                                  rbs[s], sem_in[s]).wait()
            pltpu.make_async_copy(x_hbm.at[pl.ds(base, K)], xbs[s],
                                  sem_in[s]).wait()

        def start_st(c, s):
            row0 = pl.multiple_of(base + c * K, 8)
            pltpu.async_copy(rbs[s], out_hbm.at[pl.ds(row0, K)], sem_st[s])

        def wait_st(s):
            pltpu.make_async_copy(rbs[s], out_hbm.at[pl.ds(base, K)],
                                  sem_st[s]).wait()

        def add_chunk(s):
            @pl.loop(0, K)
            def _rows(t):
                @pl.loop(0, D_MODEL // LANES, unroll=8)
                def _add(j):
                    off = pl.multiple_of(j * LANES, LANES)
                    plsc.addupdate(rbs[s].at[t, pl.ds(off, LANES)],
                                   xbs[s][t, pl.ds(off, LANES)])

        for c0 in range(LA):
            start_in(c0, c0)

        @pl.loop(0, nsteps)
        def _steps(i):
            for b in range(NB):
                c = i * NB + b
                s = b
                sn = (b + LA) % NB

                if b < LA:
                    @pl.when(i > 0)
                    def _():
                        wait_st(sn)

                    start_in(c + LA, sn)
                elif b < NB - 1:
                    wait_st(sn)
                    start_in(c + LA, sn)
                else:
                    @pl.when(i < nsteps - 1)
                    def _():
                        wait_st(sn)
                        start_in(c + LA, sn)

                wait_in(s)
                add_chunk(s)
                start_st(c, s)

        for c in range(nch - 2, nch):
            s = c % NB
            wait_in(s)
            add_chunk(s)
            start_st(c, s)

        for s in range(NB):
            wait_st(s)

    return k(x2d, tok, emb)


def kernel(x, tokens, emb_weight):
    b, l, d = x.shape
    x2d = x.reshape(b * l, d)
    tok = tokens.reshape(-1).astype(jnp.int32)
    out = _encode_sc(x2d, tok, emb_weight)
    return out.reshape(b, l, d)
